# Initial kernel scaffold; baseline (speedup 1.0000x reference)
import jax
import jax.numpy as jnp
from jax.experimental import pallas as pl


def kernel(token_ids, dense_feat, embedding_weight, linear_weight):
    raise NotImplementedError("write your pallas kernel here")


# speedup vs baseline: 2.9749x; 2.9749x over previous
import functools

import jax
import jax.numpy as jnp
from jax import lax
from jax.experimental import pallas as pl
from jax.experimental.pallas import tpu as pltpu
from jax.experimental.pallas import tpu_sc as plsc

_NC = 2
_NS = 16
_NW = _NC * _NS
_CH = 128


@functools.lru_cache(maxsize=None)
def _make_gather(V, D, B):
  b_per_w = B // _NW
  nch = b_per_w // _CH
  mesh = plsc.VectorSubcoreMesh(core_axis_name="c", subcore_axis_name="s")

  @functools.partial(
      pl.kernel,
      mesh=mesh,
      out_type=jax.ShapeDtypeStruct((B, D), jnp.float32),
      scratch_types=[
          pltpu.VMEM((nch, _CH), jnp.int32),
          pltpu.VMEM((_CH, D), jnp.float32),
          pltpu.VMEM((_CH, D), jnp.float32),
          pltpu.SemaphoreType.DMA,
          pltpu.SemaphoreType.DMA,
      ],
  )
  def gather(table_hbm, idx_hbm, out_hbm, idx_v, buf0, buf1, sem0, sem1):
    wid = lax.axis_index("s") * _NC + lax.axis_index("c")
    base = wid * b_per_w
    pltpu.sync_copy(idx_hbm.at[wid], idx_v)

    def body(i, carry):
      j0 = 2 * i
      j1 = j0 + 1
      pltpu.async_copy(table_hbm.at[idx_v.at[j0]], buf0, sem0).wait()
      pltpu.sync_copy(buf0, out_hbm.at[pl.ds(base + j0 * _CH, _CH)])
      pltpu.async_copy(table_hbm.at[idx_v.at[j1]], buf1, sem1).wait()
      pltpu.sync_copy(buf1, out_hbm.at[pl.ds(base + j1 * _CH, _CH)])
      return carry

    lax.fori_loop(0, nch // 2, body, 0)

  return gather


def _linear_tc(x, w):
  def mm(x_ref, w_ref, o_ref):
    o_ref[...] = lax.dot_general(
        x_ref[...], w_ref[...], (((1,), (1,)), ((), ())),
        preferred_element_type=jnp.float32)

  return pl.pallas_call(
      mm,
      out_shape=jax.ShapeDtypeStruct((x.shape[0], w.shape[0]), jnp.float32),
  )(x, w)


def kernel(token_ids, dense_feat, embedding_weight, linear_weight):
  B, S = token_ids.shape
  V, D = embedding_weight.shape
  n = B * S
  idx = token_ids.reshape(-1).astype(jnp.int32)
  idx3 = idx.reshape(_NW, n // (_NW * _CH), _CH)
  emb_flat = _make_gather(V, D, n)(embedding_weight, idx3)
  emb_out = emb_flat.reshape(B, S, D)
  lin_out = _linear_tc(dense_feat.astype(jnp.float32),
                       linear_weight.astype(jnp.float32))
  return (emb_out, lin_out)


# speedup vs baseline: 3.3315x; 1.1199x over previous
import functools

import jax
import jax.numpy as jnp
from jax import lax
from jax.experimental import pallas as pl
from jax.experimental.pallas import tpu as pltpu
from jax.experimental.pallas import tpu_sc as plsc

_NC = 2
_NS = 16
_NW = _NC * _NS
_CH = 128


@functools.lru_cache(maxsize=None)
def _make_gather(V, D, B):
  b_per_w = B // _NW
  nch = b_per_w // _CH
  mesh = plsc.VectorSubcoreMesh(core_axis_name="c", subcore_axis_name="s")

  @functools.partial(
      pl.kernel,
      mesh=mesh,
      out_type=jax.ShapeDtypeStruct((B, D), jnp.float32),
      scratch_types=[
          pltpu.VMEM((nch, _CH), jnp.int32),
          pltpu.VMEM((_CH, D), jnp.float32),
          pltpu.VMEM((_CH, D), jnp.float32),
          pltpu.SemaphoreType.DMA,
          pltpu.SemaphoreType.DMA,
      ],
  )
  def gather(table_hbm, idx_hbm, out_hbm, idx_v, buf0, buf1, sem0, sem1):
    wid = lax.axis_index("s") * _NC + lax.axis_index("c")
    base = wid * b_per_w
    pltpu.sync_copy(idx_hbm.at[wid], idx_v)

    def g(j, buf, sem):
      return pltpu.make_async_copy(table_hbm.at[idx_v.at[j]], buf, sem)

    g(0, buf0, sem0).start()

    def body(i, carry):
      j0 = 2 * i
      g(j0 + 1, buf1, sem1).start()
      g(j0, buf0, sem0).wait()
      pltpu.sync_copy(buf0, out_hbm.at[pl.ds(base + j0 * _CH, _CH)])

      @pl.when(j0 + 2 < nch)
      def _():
        g(j0 + 2, buf0, sem0).start()

      g(j0 + 1, buf1, sem1).wait()
      pltpu.sync_copy(buf1, out_hbm.at[pl.ds(base + (j0 + 1) * _CH, _CH)])
      return carry

    lax.fori_loop(0, nch // 2, body, 0)

  return gather


def _linear_tc(x, w):
  def mm(x_ref, w_ref, o_ref):
    o_ref[...] = lax.dot_general(
        x_ref[...], w_ref[...], (((1,), (1,)), ((), ())),
        preferred_element_type=jnp.float32)

  return pl.pallas_call(
      mm,
      out_shape=jax.ShapeDtypeStruct((x.shape[0], w.shape[0]), jnp.float32),
  )(x, w)


def kernel(token_ids, dense_feat, embedding_weight, linear_weight):
  B, S = token_ids.shape
  V, D = embedding_weight.shape
  n = B * S
  idx = token_ids.reshape(-1).astype(jnp.int32)
  idx3 = idx.reshape(_NW, n // (_NW * _CH), _CH)
  emb_flat = _make_gather(V, D, n)(embedding_weight, idx3)
  emb_out = emb_flat.reshape(B, S, D)
  lin_out = _linear_tc(dense_feat.astype(jnp.float32),
                       linear_weight.astype(jnp.float32))
  return (emb_out, lin_out)
